# Initial kernel scaffold; baseline (speedup 1.0000x reference)
import jax
import jax.numpy as jnp
from jax.experimental import pallas as pl


def kernel(x, pos_emb):
    raise NotImplementedError("write your pallas kernel here")


# speedup vs baseline: 1.4917x; 1.4917x over previous
import jax
import jax.numpy as jnp
from jax.experimental import pallas as pl


def _add_body(x_ref, p_ref, o_ref):
    o_ref[...] = x_ref[...] + p_ref[...]


def kernel(x, pos_emb):
    B, T, C = x.shape
    BT = 512
    grid = (T // BT, B)
    return pl.pallas_call(
        _add_body,
        grid=grid,
        in_specs=[
            pl.BlockSpec((1, BT, C), lambda i, j: (j, i, 0)),
            pl.BlockSpec((BT, C), lambda i, j: (i, 0)),
        ],
        out_specs=pl.BlockSpec((1, BT, C), lambda i, j: (j, i, 0)),
        out_shape=jax.ShapeDtypeStruct((B, T, C), x.dtype),
    )(x, pos_emb)


# speedup vs baseline: 1.6539x; 1.1088x over previous
import jax
import jax.numpy as jnp
from jax.experimental import pallas as pl


def _add_body(x_ref, p_ref, o_ref):
    o_ref[...] = x_ref[...] + p_ref[...]


def kernel(x, pos_emb):
    B, T, C = x.shape
    BT = 1024
    grid = (T // BT, B)
    return pl.pallas_call(
        _add_body,
        grid=grid,
        in_specs=[
            pl.BlockSpec((1, BT, C), lambda i, j: (j, i, 0)),
            pl.BlockSpec((BT, C), lambda i, j: (i, 0)),
        ],
        out_specs=pl.BlockSpec((1, BT, C), lambda i, j: (j, i, 0)),
        out_shape=jax.ShapeDtypeStruct((B, T, C), x.dtype),
    )(x, pos_emb)


# speedup vs baseline: 1.7376x; 1.0506x over previous
import jax
import jax.numpy as jnp
from jax.experimental import pallas as pl


def _add_body(x_ref, p_ref, o_ref):
    o_ref[...] = x_ref[...] + p_ref[...]


def kernel(x, pos_emb):
    B, T, C = x.shape
    BT = 2048
    grid = (T // BT, B)
    return pl.pallas_call(
        _add_body,
        grid=grid,
        in_specs=[
            pl.BlockSpec((1, BT, C), lambda i, j: (j, i, 0)),
            pl.BlockSpec((BT, C), lambda i, j: (i, 0)),
        ],
        out_specs=pl.BlockSpec((1, BT, C), lambda i, j: (j, i, 0)),
        out_shape=jax.ShapeDtypeStruct((B, T, C), x.dtype),
    )(x, pos_emb)
